# Initial kernel scaffold; baseline (speedup 1.0000x reference)
import jax
import jax.numpy as jnp
from jax.experimental import pallas as pl


def kernel(x_drug, x_prot, edge_index_dp, edge_index_pd, edge_label_index, Wl_dp, Wr_dp, b_dp, Wl_pd, Wr_pd, b_pd, W_lin, b_lin):
    raise NotImplementedError("write your pallas kernel here")


# speedup vs baseline: 23.7590x; 23.7590x over previous
import functools

import jax
import jax.numpy as jnp
from jax import lax
from jax.experimental import pallas as pl
from jax.experimental.pallas import tpu as pltpu
from jax.experimental.pallas import tpu_sc as plsc

N = 10000
NPAD = 10240
E = 320000
D = 128
L = 100000
LPAD = 100352
NW = 32
EPW = E // NW
NPW = NPAD // NW
LPW = LPAD // NW
LANES = 16

f32 = jnp.float32
i32 = jnp.int32


def _dense_body(xd, xp, wldp, wrdp, wlpd, wrpd, wlin, bdp, bpd, blin,
                t_dp, t_pd, self_d, self_p):
    w1 = wlin[0:D, 0]
    w2 = wlin[D:2 * D, 0]
    v_dp = jnp.sum(wldp[...] * w2[None, :], axis=1)
    u_dp = jnp.sum(wrdp[...] * w2[None, :], axis=1)
    v_pd = jnp.sum(wlpd[...] * w1[None, :], axis=1)
    u_pd = jnp.sum(wrpd[...] * w1[None, :], axis=1)
    c_prot = jnp.sum(bdp[...] * w2)
    c_drug = jnp.sum(bpd[...] * w1) + jnp.sum(blin[...])
    xdv = xd[...]
    xpv = xp[...]
    t_dp[...] = jnp.sum(xdv * v_dp[None, :], axis=1)
    t_pd[...] = jnp.sum(xpv * v_pd[None, :], axis=1)
    self_d[...] = jnp.sum(xdv * u_pd[None, :], axis=1) + c_drug
    self_p[...] = jnp.sum(xpv * u_dp[None, :], axis=1) + c_prot


_dense = pl.pallas_call(
    _dense_body,
    out_shape=[jax.ShapeDtypeStruct((N,), f32)] * 4,
)


_MESH = plsc.VectorSubcoreMesh(core_axis_name="c", subcore_axis_name="s",
                               num_cores=2, num_subcores=16)
_SC_PARAMS = pltpu.CompilerParams(needs_layout_passes=False,
                                  use_tc_tiling_on_sc=False)


def _wid():
    return lax.axis_index("s") * 2 + lax.axis_index("c")


@functools.partial(
    pl.kernel,
    out_type=[jax.ShapeDtypeStruct((NW, NPAD), f32)] * 4,
    mesh=_MESH,
    compiler_params=_SC_PARAMS,
    scratch_types=[
        pltpu.VMEM((EPW,), i32),
        pltpu.VMEM((EPW,), i32),
        pltpu.VMEM((N,), f32),
        pltpu.VMEM((NPAD,), f32),
        pltpu.VMEM((NPAD,), f32),
        pltpu.VMEM((LANES,), i32),
    ],
)
def _segsum(src_dp, dst_dp, src_pd, dst_pd, t_dp, t_pd,
            acc_dp_o, cnt_dp_o, acc_pd_o, cnt_pd_o,
            src_v, dst_v, t_v, acc_v, cnt_v, shift_v):
    w = _wid()
    iota = lax.iota(i32, LANES)
    iota_f = iota.astype(f32)
    shift_idx = jnp.minimum(iota + 1, LANES - 1)
    last = iota == LANES - 1
    zeros = jnp.zeros((LANES,), f32)

    def relation(src_h, dst_h, t_h, acc_o, cnt_o):
        pltpu.sync_copy(t_h, t_v)
        pltpu.sync_copy(src_h.at[pl.ds(w * EPW, EPW)], src_v)
        pltpu.sync_copy(dst_h.at[pl.ds(w * EPW, EPW)], dst_v)

        def zero_body(i, _):
            acc_v[pl.ds(i * LANES, LANES)] = zeros
            cnt_v[pl.ds(i * LANES, LANES)] = zeros
            return 0
        lax.fori_loop(0, NPAD // LANES, zero_body, 0)

        def body(i, _):
            d16 = dst_v[pl.ds(i * LANES, LANES)]
            s16 = src_v[pl.ds(i * LANES, LANES)]
            sk, sv = plsc.sort_key_val(d16, s16)
            vals = plsc.load_gather(t_v, [sv])
            csum = plsc.cumsum(vals)
            shift_v[...] = sk
            nxt = plsc.load_gather(shift_v, [shift_idx])
            end = last | (sk != nxt)
            endn = end & jnp.logical_not(last)
            cf = iota_f + 1.0
            plsc.addupdate_scatter(acc_v, [sk], csum, mask=end)
            plsc.addupdate_scatter(acc_v, [nxt], -csum, mask=endn)
            plsc.addupdate_scatter(cnt_v, [sk], cf, mask=end)
            plsc.addupdate_scatter(cnt_v, [nxt], -cf, mask=endn)
            return 0
        lax.fori_loop(0, EPW // LANES, body, 0)
        pltpu.sync_copy(acc_v, acc_o.at[w])
        pltpu.sync_copy(cnt_v, cnt_o.at[w])

    relation(src_dp, dst_dp, t_dp, acc_dp_o, cnt_dp_o)
    relation(src_pd, dst_pd, t_pd, acc_pd_o, cnt_pd_o)


@functools.partial(
    pl.kernel,
    out_type=[jax.ShapeDtypeStruct((NPAD,), f32)] * 2,
    mesh=_MESH,
    compiler_params=_SC_PARAMS,
    scratch_types=[
        pltpu.VMEM((NW, NPW), f32),
        pltpu.VMEM((NPW,), f32),
        pltpu.VMEM((NPW,), f32),
        pltpu.VMEM((NPW,), f32),
    ],
)
def _finalize(acc_dp, cnt_dp, acc_pd, cnt_pd, self_p, self_d,
              s_prot_o, s_drug_o,
              part_v, sum_v, self_v, out_v):
    w = _wid()
    nb = w * NPW
    zeros = jnp.zeros((LANES,), f32)

    def side(acc_h, cnt_h, self_h, s_o):
        pltpu.sync_copy(acc_h.at[:, pl.ds(nb, NPW)], part_v)

        def sum_body(c, _):
            v = zeros
            for r in range(NW):
                v = v + part_v[r, pl.ds(c * LANES, LANES)]
            sum_v[pl.ds(c * LANES, LANES)] = v
            return 0
        lax.fori_loop(0, NPW // LANES, sum_body, 0)

        pltpu.sync_copy(cnt_h.at[:, pl.ds(nb, NPW)], part_v)
        pltpu.sync_copy(self_h.at[pl.ds(nb, NPW)], self_v)

        def fin_body(c, _):
            sl = pl.ds(c * LANES, LANES)
            cv = zeros
            for r in range(NW):
                cv = cv + part_v[r, sl]
            out_v[sl] = sum_v[sl] / jnp.maximum(cv, 1.0) + self_v[sl]
            return 0
        lax.fori_loop(0, NPW // LANES, fin_body, 0)
        pltpu.sync_copy(out_v, s_o.at[pl.ds(nb, NPW)])

    side(acc_dp, cnt_dp, self_p, s_prot_o)
    side(acc_pd, cnt_pd, self_d, s_drug_o)


@functools.partial(
    pl.kernel,
    out_type=jax.ShapeDtypeStruct((LPAD,), f32),
    mesh=_MESH,
    compiler_params=_SC_PARAMS,
    scratch_types=[
        pltpu.VMEM((NPAD,), f32),
        pltpu.VMEM((NPAD,), f32),
        pltpu.VMEM((LPW,), i32),
        pltpu.VMEM((LPW,), i32),
        pltpu.VMEM((LPW,), f32),
    ],
)
def _edge_gather(s_drug, s_prot, eli0, eli1, out_o,
                 sd_v, sp_v, e0_v, e1_v, o_v):
    w = _wid()
    lb = w * LPW
    pltpu.sync_copy(s_drug, sd_v)
    pltpu.sync_copy(s_prot, sp_v)
    pltpu.sync_copy(eli0.at[pl.ds(lb, LPW)], e0_v)
    pltpu.sync_copy(eli1.at[pl.ds(lb, LPW)], e1_v)

    def body(i, _):
        sl = pl.ds(i * LANES, LANES)
        o_v[sl] = (plsc.load_gather(sd_v, [e0_v[sl]])
                   + plsc.load_gather(sp_v, [e1_v[sl]]))
        return 0
    lax.fori_loop(0, LPW // LANES, body, 0)
    pltpu.sync_copy(o_v, out_o.at[pl.ds(lb, LPW)])


def kernel(x_drug, x_prot, edge_index_dp, edge_index_pd, edge_label_index,
           Wl_dp, Wr_dp, b_dp, Wl_pd, Wr_pd, b_pd, W_lin, b_lin):
    t_dp, t_pd, self_d, self_p = _dense(
        x_drug, x_prot, Wl_dp, Wr_dp, Wl_pd, Wr_pd, W_lin, b_dp, b_pd, b_lin)

    src_dp = edge_index_dp[0].astype(i32)
    dst_dp = edge_index_dp[1].astype(i32)
    src_pd = edge_index_pd[0].astype(i32)
    dst_pd = edge_index_pd[1].astype(i32)
    eli0 = jnp.pad(edge_label_index[0].astype(i32), (0, LPAD - L))
    eli1 = jnp.pad(edge_label_index[1].astype(i32), (0, LPAD - L))
    self_d_pad = jnp.pad(self_d, (0, NPAD - N))
    self_p_pad = jnp.pad(self_p, (0, NPAD - N))

    acc_dp, cnt_dp, acc_pd, cnt_pd = _segsum(
        src_dp, dst_dp, src_pd, dst_pd, t_dp, t_pd)
    s_prot, s_drug = _finalize(
        acc_dp, cnt_dp, acc_pd, cnt_pd, self_p_pad, self_d_pad)
    out = _edge_gather(s_drug, s_prot, eli0, eli1)
    return out[:L][:, None]


# speedup vs baseline: 36.1438x; 1.5213x over previous
import functools

import jax
import jax.numpy as jnp
from jax import lax
from jax.experimental import pallas as pl
from jax.experimental.pallas import tpu as pltpu
from jax.experimental.pallas import tpu_sc as plsc

N = 10000
NPAD = 10240
E = 320000
D = 128
L = 100000
LPAD = 100352
NW = 32
EPW = E // NW
NPW = NPAD // NW
LPW = LPAD // NW
LANES = 16

f32 = jnp.float32
i32 = jnp.int32


def _dense_t_body(xd, xp, wldp, wlpd, wlin, t_dp, t_pd):
    w1 = wlin[0:D, 0]
    w2 = wlin[D:2 * D, 0]
    v_dp = jnp.sum(wldp[...] * w2[None, :], axis=1)
    v_pd = jnp.sum(wlpd[...] * w1[None, :], axis=1)
    t_dp[...] = jnp.sum(xd[...] * v_dp[None, :], axis=1)
    t_pd[...] = jnp.sum(xp[...] * v_pd[None, :], axis=1)


def _dense_self_body(xd, xp, wrdp, wrpd, wlin, bdp, bpd, blin,
                     self_d, self_p):
    w1 = wlin[0:D, 0]
    w2 = wlin[D:2 * D, 0]
    u_dp = jnp.sum(wrdp[...] * w2[None, :], axis=1)
    u_pd = jnp.sum(wrpd[...] * w1[None, :], axis=1)
    c_prot = jnp.sum(bdp[...] * w2)
    c_drug = jnp.sum(bpd[...] * w1) + jnp.sum(blin[...])
    self_d[...] = jnp.sum(xd[...] * u_pd[None, :], axis=1) + c_drug
    self_p[...] = jnp.sum(xp[...] * u_dp[None, :], axis=1) + c_prot


_dense_t = pl.pallas_call(
    _dense_t_body,
    out_shape=[jax.ShapeDtypeStruct((N,), f32)] * 2,
)

_dense_self = pl.pallas_call(
    _dense_self_body,
    out_shape=[jax.ShapeDtypeStruct((N,), f32)] * 2,
)


_MESH = plsc.VectorSubcoreMesh(core_axis_name="c", subcore_axis_name="s",
                               num_cores=2, num_subcores=16)
_SC_PARAMS = pltpu.CompilerParams(needs_layout_passes=False,
                                  use_tc_tiling_on_sc=False)


def _wid():
    return lax.axis_index("s") * 2 + lax.axis_index("c")


@functools.partial(
    pl.kernel,
    out_type=[jax.ShapeDtypeStruct((NW, NPAD), f32)] * 4,
    mesh=_MESH,
    compiler_params=_SC_PARAMS,
    scratch_types=[
        pltpu.VMEM((EPW,), i32),
        pltpu.VMEM((EPW,), i32),
        pltpu.VMEM((N,), f32),
        pltpu.VMEM((NPAD,), f32),
        pltpu.VMEM((NPAD,), f32),
    ],
)
def _segsum(ei_dp, ei_pd, t_dp, t_pd,
            acc_dp_o, cnt_dp_o, acc_pd_o, cnt_pd_o,
            src_v, dst_v, t_v, acc_v, cnt_v):
    w = _wid()
    zeros = jnp.zeros((LANES,), f32)
    ones = jnp.full((LANES,), 1.0, f32)

    def relation(ei_h, t_h, acc_o, cnt_o):
        pltpu.sync_copy(t_h, t_v)
        pltpu.sync_copy(ei_h.at[0, pl.ds(w * EPW, EPW)], src_v)
        pltpu.sync_copy(ei_h.at[1, pl.ds(w * EPW, EPW)], dst_v)

        def zero_body(i, _):
            acc_v[pl.ds(i * LANES, LANES)] = zeros
            cnt_v[pl.ds(i * LANES, LANES)] = zeros
            return 0
        lax.fori_loop(0, NPAD // LANES, zero_body, 0)

        def body(i, _):
            sl = pl.ds(i * LANES, LANES)
            d16 = dst_v[sl]
            vals = plsc.load_gather(t_v, [src_v[sl]])
            plsc.addupdate_scatter(acc_v, [d16], vals)
            plsc.addupdate_scatter(cnt_v, [d16], ones)
            return 0
        lax.fori_loop(0, EPW // LANES, body, 0)
        pltpu.sync_copy(acc_v, acc_o.at[w])
        pltpu.sync_copy(cnt_v, cnt_o.at[w])

    relation(ei_dp, t_dp, acc_dp_o, cnt_dp_o)
    relation(ei_pd, t_pd, acc_pd_o, cnt_pd_o)


@functools.partial(
    pl.kernel,
    out_type=[jax.ShapeDtypeStruct((NPAD,), f32)] * 2,
    mesh=_MESH,
    compiler_params=_SC_PARAMS,
    scratch_types=[
        pltpu.VMEM((NW, NPW), f32),
        pltpu.VMEM((NPW,), f32),
        pltpu.VMEM((NPW,), f32),
        pltpu.VMEM((NPW,), f32),
    ],
)
def _finalize(acc_dp, cnt_dp, acc_pd, cnt_pd, self_p, self_d,
              s_prot_o, s_drug_o,
              part_v, sum_v, self_v, out_v):
    w = _wid()
    nb = w * NPW
    zeros = jnp.zeros((LANES,), f32)

    def side(acc_h, cnt_h, self_h, s_o):
        pltpu.sync_copy(acc_h.at[:, pl.ds(nb, NPW)], part_v)

        def sum_body(c, _):
            v = zeros
            for r in range(NW):
                v = v + part_v[r, pl.ds(c * LANES, LANES)]
            sum_v[pl.ds(c * LANES, LANES)] = v
            return 0
        lax.fori_loop(0, NPW // LANES, sum_body, 0)

        pltpu.sync_copy(cnt_h.at[:, pl.ds(nb, NPW)], part_v)
        pltpu.sync_copy(self_h.at[pl.ds(nb, NPW)], self_v)

        def fin_body(c, _):
            sl = pl.ds(c * LANES, LANES)
            cv = zeros
            for r in range(NW):
                cv = cv + part_v[r, sl]
            out_v[sl] = sum_v[sl] / jnp.maximum(cv, 1.0) + self_v[sl]
            return 0
        lax.fori_loop(0, NPW // LANES, fin_body, 0)
        pltpu.sync_copy(out_v, s_o.at[pl.ds(nb, NPW)])

    side(acc_dp, cnt_dp, self_p, s_prot_o)
    side(acc_pd, cnt_pd, self_d, s_drug_o)


@functools.partial(
    pl.kernel,
    out_type=jax.ShapeDtypeStruct((LPAD,), f32),
    mesh=_MESH,
    compiler_params=_SC_PARAMS,
    scratch_types=[
        pltpu.VMEM((NPAD,), f32),
        pltpu.VMEM((NPAD,), f32),
        pltpu.VMEM((LPW,), i32),
        pltpu.VMEM((LPW,), i32),
        pltpu.VMEM((LPW,), f32),
    ],
)
def _edge_gather(s_drug, s_prot, eli0, eli1, out_o,
                 sd_v, sp_v, e0_v, e1_v, o_v):
    w = _wid()
    lb = w * LPW
    pltpu.sync_copy(s_drug, sd_v)
    pltpu.sync_copy(s_prot, sp_v)
    pltpu.sync_copy(eli0.at[pl.ds(lb, LPW)], e0_v)
    pltpu.sync_copy(eli1.at[pl.ds(lb, LPW)], e1_v)

    def body(i, _):
        sl = pl.ds(i * LANES, LANES)
        o_v[sl] = (plsc.load_gather(sd_v, [e0_v[sl]])
                   + plsc.load_gather(sp_v, [e1_v[sl]]))
        return 0
    lax.fori_loop(0, LPW // LANES, body, 0)
    pltpu.sync_copy(o_v, out_o.at[pl.ds(lb, LPW)])


def kernel(x_drug, x_prot, edge_index_dp, edge_index_pd, edge_label_index,
           Wl_dp, Wr_dp, b_dp, Wl_pd, Wr_pd, b_pd, W_lin, b_lin):
    t_dp, t_pd = _dense_t(x_drug, x_prot, Wl_dp, Wl_pd, W_lin)
    self_d, self_p = _dense_self(
        x_drug, x_prot, Wr_dp, Wr_pd, W_lin, b_dp, b_pd, b_lin)

    eli0 = jnp.pad(edge_label_index[0].astype(i32), (0, LPAD - L))
    eli1 = jnp.pad(edge_label_index[1].astype(i32), (0, LPAD - L))
    self_d_pad = jnp.pad(self_d, (0, NPAD - N))
    self_p_pad = jnp.pad(self_p, (0, NPAD - N))

    acc_dp, cnt_dp, acc_pd, cnt_pd = _segsum(
        edge_index_dp.astype(i32), edge_index_pd.astype(i32), t_dp, t_pd)
    s_prot, s_drug = _finalize(
        acc_dp, cnt_dp, acc_pd, cnt_pd, self_p_pad, self_d_pad)
    out = _edge_gather(s_drug, s_prot, eli0, eli1)
    return out[:L][:, None]
